# baseline (speedup 1.0000x reference)
import functools

import jax
import jax.numpy as jnp
from jax import lax
from jax.experimental import pallas as pl
from jax.experimental.pallas import tpu as pltpu
from jax.experimental.pallas import tpu_sc as plsc

_D = 64
_T = 20
_BATCH = 16384
_B = _BATCH * _T
_NC, _NS = 2, 16
_NW = _NC * _NS
_CHUNK = 128
_NBLK = _B // _CHUNK
_PER_W = _NBLK // _NW
_CTILE = _BATCH // _CHUNK
_NB = 8
_LOOKAHEAD = 6

_mesh = plsc.VectorSubcoreMesh(core_axis_name="c", subcore_axis_name="s")


@functools.partial(
    pl.kernel,
    out_type=jax.ShapeDtypeStruct((_T, _D // 8, _CTILE, 8, _CHUNK), jnp.float32),
    mesh=_mesh,
    scratch_types=[
        pltpu.VMEM((_NB, _CHUNK), jnp.int32),
        pltpu.VMEM((_NB, _CHUNK, _D), jnp.float32),
        pltpu.VMEM((2, _D // 8, 8, _CHUNK), jnp.float32),
        [pltpu.SemaphoreType.DMA] * _NB,
        [pltpu.SemaphoreType.DMA] * _NB,
        [pltpu.SemaphoreType.DMA] * 2,
    ],
    compiler_params=pltpu.CompilerParams(
        use_tc_tiling_on_sc=False, needs_layout_passes=False
    ),
)
def _emb_lookup(idx_hbm, table_hbm, out_hbm, idx_v, rows_v, tr_v, isems, gsems,
                wsems):
    wid = lax.axis_index("s") * _NC + lax.axis_index("c")

    def issue_idx(j, b):
        blk = wid * _PER_W + j
        t = blk // _CTILE
        c = blk % _CTILE
        pltpu.async_copy(idx_hbm.at[t].at[c], idx_v.at[b], isems[b])

    def issue_gather(j, b):
        del j
        pltpu.make_async_copy(
            idx_hbm.at[0].at[0], idx_v.at[b], isems[b]
        ).wait()
        pltpu.async_copy(table_hbm.at[idx_v.at[b]], rows_v.at[b], gsems[b])

    for p in range(_LOOKAHEAD + 1):
        issue_idx(p, p)
    for p in range(_LOOKAHEAD):
        issue_gather(p, p)

    def body(jj, carry):
        for b in range(_NB):
            j = jj * _NB + b
            blk = wid * _PER_W + j
            t = blk // _CTILE
            c = blk % _CTILE
            jg = j + _LOOKAHEAD

            @pl.when(jg < _PER_W)
            def _():
                issue_gather(jg, (b + _LOOKAHEAD) % _NB)

            ji = j + _LOOKAHEAD + 1

            @pl.when(ji < _PER_W)
            def _():
                issue_idx(ji, (b + _LOOKAHEAD + 1) % _NB)

            pltpu.make_async_copy(
                table_hbm.at[pl.ds(0, _CHUNK)], rows_v.at[b], gsems[b]
            ).wait()
            rows_b = rows_v.at[b]
            w = b % 2
            tr_w = tr_v.at[w]

            @pl.when(j >= 2)
            def _():
                pltpu.make_async_copy(
                    out_hbm.at[0].at[:, 0], tr_w, wsems[w]
                ).wait()

            def dloop(dt, dcarry):
                for ds_ in range(8):
                    d = dt * 8 + ds_
                    cidx = jnp.full((16,), 0, jnp.int32) + d
                    grp = []
                    for bl0 in range(0, _CHUNK, 16):
                        ridx = bl0 + lax.iota(jnp.int32, 16)
                        grp.append(plsc.load_gather(rows_b, [ridx, cidx]))
                    for gi, bl0 in enumerate(range(0, _CHUNK, 16)):
                        tr_w[dt, ds_, pl.ds(bl0, 16)] = grp[gi]
                return dcarry

            lax.fori_loop(0, _D // 8, dloop, 0)
            pltpu.async_copy(tr_w, out_hbm.at[t].at[:, c], wsems[w])
        return carry

    lax.fori_loop(0, _PER_W // _NB, body, 0)
    for w in range(2):
        pltpu.make_async_copy(
            out_hbm.at[0].at[:, 0], tr_v.at[w], wsems[w]
        ).wait()


def kernel(x, vocab):
    idx = x.T.reshape(_T, _CTILE, _CHUNK)
    out5 = _emb_lookup(idx, vocab)
    out = jnp.transpose(out5, (2, 4, 0, 1, 3)).reshape(_BATCH, _T, _D)
    return out


# speedup vs baseline: 1.1406x; 1.1406x over previous
import functools

import jax
import jax.numpy as jnp
from jax import lax
from jax.experimental import pallas as pl
from jax.experimental.pallas import tpu as pltpu
from jax.experimental.pallas import tpu_sc as plsc

_D = 64
_B = 16384 * 20
_NC, _NS = 2, 16
_NW = _NC * _NS
_CHUNK = 128
_PER_W = _B // _NW
_NCHUNK = _PER_W // _CHUNK

_mesh = plsc.VectorSubcoreMesh(core_axis_name="c", subcore_axis_name="s")


_K = 4
_GROUP = _K * _CHUNK
_NGROUP = _PER_W // _GROUP
_NB = 2


@functools.partial(
    pl.kernel,
    out_type=jax.ShapeDtypeStruct((_B, _D), jnp.float32),
    mesh=_mesh,
    scratch_types=[
        pltpu.VMEM((_NCHUNK, _CHUNK), jnp.int32),
        pltpu.VMEM((_NB, _GROUP, _D), jnp.float32),
        pltpu.SemaphoreType.DMA,
        pltpu.SemaphoreType.DMA,
    ],
    compiler_params=pltpu.CompilerParams(use_tc_tiling_on_sc=False),
)
def _emb_lookup(idx_hbm, table_hbm, out_hbm, idx_v, rows_v, sem0, sem1):
    wid = lax.axis_index("s") * _NC + lax.axis_index("c")
    sems = (sem0, sem1)
    pltpu.sync_copy(idx_hbm.at[pl.ds(wid * _NCHUNK, _NCHUNK)], idx_v)

    def issue_group(g, b):
        for k in range(_K):
            pltpu.async_copy(
                table_hbm.at[idx_v.at[g * _K + k]],
                rows_v.at[b].at[pl.ds(k * _CHUNK, _CHUNK)],
                sems[b],
            )

    for b in range(_NB):
        issue_group(b, b)

    def body(gg, carry):
        for b in range(_NB):
            g = gg * _NB + b
            pltpu.make_async_copy(
                table_hbm.at[pl.ds(0, _GROUP)], rows_v.at[b], sems[b]
            ).wait()
            row0 = wid * _PER_W + g * _GROUP
            pltpu.sync_copy(rows_v.at[b], out_hbm.at[pl.ds(row0, _GROUP)])
            gn = g + _NB

            @pl.when(gn < _NGROUP)
            def _():
                issue_group(gn, b)

        return carry

    lax.fori_loop(0, _NGROUP // _NB, body, 0)


def kernel(x, vocab):
    idx = x.reshape(_B // _CHUNK, _CHUNK)
    out = _emb_lookup(idx, vocab)
    return out.reshape(x.shape + (_D,))
